# baseline (speedup 1.0000x reference)
import functools

import jax
import jax.numpy as jnp
from jax import lax
from jax.experimental import pallas as pl
from jax.experimental.pallas import tpu as pltpu
from jax.experimental.pallas import tpu_sc as plsc

N = 10000
E = 320000
DIN = 128
DH = 128
DL = 64
DOUT = 128
DE = 16
G = 64

NP_ = 10240
EP = 327680
NTILES = 32
EPT = EP // NTILES
CH = 128
NCH = EPT // CH
BN = 1024
RPT = NP_ // 16


def _node1_body(x_ref, w_ref, asrc_ref, adst_ref, h_ref, ssd_ref):
    xw = jnp.dot(x_ref[...], w_ref[...], preferred_element_type=jnp.float32)
    h_ref[...] = xw
    s = jnp.sum(xw * asrc_ref[...], axis=1).reshape(1, BN)
    d = jnp.sum(xw * adst_ref[...], axis=1).reshape(1, BN)
    ssd_ref[...] = jnp.concatenate(
        [s, d, jnp.zeros((6, BN), jnp.float32)], axis=0)


_node1 = pl.pallas_call(
    _node1_body,
    grid=(NP_ // BN,),
    in_specs=[
        pl.BlockSpec((BN, DIN), lambda j: (j, 0)),
        pl.BlockSpec((DIN, DH), lambda j: (0, 0)),
        pl.BlockSpec((1, DH), lambda j: (0, 0)),
        pl.BlockSpec((1, DH), lambda j: (0, 0)),
    ],
    out_specs=[
        pl.BlockSpec((BN, DH), lambda j: (j, 0)),
        pl.BlockSpec((8, BN), lambda j: (0, j)),
    ],
    out_shape=[
        jax.ShapeDtypeStruct((NP_, DH), jnp.float32),
        jax.ShapeDtypeStruct((8, NP_), jnp.float32),
    ],
)

BE = 4096


def _edgefeat_body(ea_ref, we1_ref, ae1_ref, we2_ref, ae2_ref, e12_ref):
    ea = ea_ref[...]
    ve1 = jnp.sum(we1_ref[...] * ae1_ref[...], axis=1)
    ve2 = jnp.sum(we2_ref[...] * ae2_ref[...], axis=1)
    e1 = jnp.sum(ea * ve1[None, :], axis=1).reshape(1, BE)
    e2 = jnp.sum(ea * ve2[None, :], axis=1).reshape(1, BE)
    e12_ref[...] = jnp.concatenate(
        [e1, e2, jnp.zeros((6, BE), jnp.float32)], axis=0)


_edgefeat = pl.pallas_call(
    _edgefeat_body,
    grid=(EP // BE,),
    in_specs=[
        pl.BlockSpec((BE, DE), lambda j: (j, 0)),
        pl.BlockSpec((DE, DH), lambda j: (0, 0)),
        pl.BlockSpec((1, DH), lambda j: (0, 0)),
        pl.BlockSpec((DE, DL), lambda j: (0, 0)),
        pl.BlockSpec((1, DL), lambda j: (0, 0)),
    ],
    out_specs=[pl.BlockSpec((8, BE), lambda j: (0, j))],
    out_shape=[jax.ShapeDtypeStruct((8, EP), jnp.float32)],
)


def _combine2_body(op_ref, dp_ref, b1_ref, w2_ref, as2_ref, ad2_ref,
                   h2_ref, ssd_ref):
    agg = op_ref[0] + op_ref[1]
    den = jnp.sum(dp_ref[...], axis=0)
    hr = jnp.maximum(agg / (den[:, None] + 1e-16) + b1_ref[...], 0.0)
    h2 = jnp.dot(hr, w2_ref[...], preferred_element_type=jnp.float32)
    h2_ref[...] = jnp.concatenate(
        [h2, jnp.zeros((BN, DH - DL), jnp.float32)], axis=1)
    s = jnp.sum(h2 * as2_ref[...], axis=1).reshape(1, BN)
    d = jnp.sum(h2 * ad2_ref[...], axis=1).reshape(1, BN)
    ssd_ref[...] = jnp.concatenate(
        [s, d, jnp.zeros((6, BN), jnp.float32)], axis=0)


_combine2 = pl.pallas_call(
    _combine2_body,
    grid=(NP_ // BN,),
    in_specs=[
        pl.BlockSpec((2, BN, DH), lambda j: (0, j, 0)),
        pl.BlockSpec((NTILES, BN), lambda j: (0, j)),
        pl.BlockSpec((1, DH), lambda j: (0, 0)),
        pl.BlockSpec((DH, DL), lambda j: (0, 0)),
        pl.BlockSpec((1, DL), lambda j: (0, 0)),
        pl.BlockSpec((1, DL), lambda j: (0, 0)),
    ],
    out_specs=[
        pl.BlockSpec((BN, DH), lambda j: (j, 0)),
        pl.BlockSpec((8, BN), lambda j: (0, j)),
    ],
    out_shape=[
        jax.ShapeDtypeStruct((NP_, DH), jnp.float32),
        jax.ShapeDtypeStruct((8, NP_), jnp.float32),
    ],
)


def _final_body(op_ref, dp_ref, b2_ref, wl_ref, bl_ref, batch_ref,
                wd1_ref, bd1_ref, wd2_ref, bd2_ref,
                z_ref, xh_ref, ge_ref, pool_ref):
    j = pl.program_id(0)
    agg = (op_ref[0] + op_ref[1])[:, 0:DL]
    den = jnp.sum(dp_ref[...], axis=0)
    zin = agg / (den[:, None] + 1e-16) + b2_ref[...]
    z = jnp.dot(zin, wl_ref[...], preferred_element_type=jnp.float32) + bl_ref[...]
    z_ref[...] = z
    b = batch_ref[0, 0, :]
    onehot = (b[:, None] == lax.broadcasted_iota(jnp.int32, (1, G), 1)
              ).astype(jnp.float32)
    gs = lax.dot_general(onehot, z, (((0,), (0,)), ((), ())),
                         preferred_element_type=jnp.float32)
    cnt = jnp.sum(onehot, axis=0).reshape(G, 1)

    @pl.when(j == 0)
    def _():
        pool_ref[...] = jnp.zeros((G, 128), jnp.float32)

    pool_ref[:, 0:DL] = pool_ref[:, 0:DL] + gs
    pool_ref[:, DL:DL + 1] = pool_ref[:, DL:DL + 1] + cnt

    hd = jnp.maximum(
        jnp.dot(z, wd1_ref[...], preferred_element_type=jnp.float32)
        + bd1_ref[...], 0.0)
    xh_ref[...] = (jnp.dot(hd, wd2_ref[...], preferred_element_type=jnp.float32)
                   + bd2_ref[...])

    @pl.when(j == NP_ // BN - 1)
    def _():
        ge_ref[...] = pool_ref[:, 0:DL] / jnp.maximum(
            pool_ref[:, DL:DL + 1], 1.0)


_final = pl.pallas_call(
    _final_body,
    grid=(NP_ // BN,),
    in_specs=[
        pl.BlockSpec((2, BN, DH), lambda j: (0, j, 0)),
        pl.BlockSpec((NTILES, BN), lambda j: (0, j)),
        pl.BlockSpec((1, DL), lambda j: (0, 0)),
        pl.BlockSpec((DL, DL), lambda j: (0, 0)),
        pl.BlockSpec((1, DL), lambda j: (0, 0)),
        pl.BlockSpec((1, 1, BN), lambda j: (j, 0, 0)),
        pl.BlockSpec((DL, DH), lambda j: (0, 0)),
        pl.BlockSpec((1, DH), lambda j: (0, 0)),
        pl.BlockSpec((DH, DOUT), lambda j: (0, 0)),
        pl.BlockSpec((1, DOUT), lambda j: (0, 0)),
    ],
    out_specs=[
        pl.BlockSpec((BN, DL), lambda j: (j, 0)),
        pl.BlockSpec((BN, DOUT), lambda j: (j, 0)),
        pl.BlockSpec((G, DL), lambda j: (0, 0)),
    ],
    out_shape=[
        jax.ShapeDtypeStruct((NP_, DL), jnp.float32),
        jax.ShapeDtypeStruct((NP_, DOUT), jnp.float32),
        jax.ShapeDtypeStruct((G, DL), jnp.float32),
    ],
    scratch_shapes=[pltpu.VMEM((G, 128), jnp.float32)],
)


_sc_mesh = plsc.VectorSubcoreMesh(core_axis_name="c", subcore_axis_name="s")


@functools.partial(
    pl.kernel,
    out_type=[
        jax.ShapeDtypeStruct((NTILES, NCH, CH), jnp.float32),
        jax.ShapeDtypeStruct((NTILES, NP_), jnp.float32),
    ],
    mesh=_sc_mesh,
    compiler_params=pltpu.CompilerParams(needs_layout_passes=False),
    scratch_types=[
        pltpu.VMEM((NP_,), jnp.float32),
        pltpu.VMEM((NP_,), jnp.float32),
        pltpu.VMEM((NP_,), jnp.float32),
        pltpu.VMEM((NCH, CH), jnp.int32),
        pltpu.VMEM((NCH, CH), jnp.int32),
        pltpu.VMEM((NCH, CH), jnp.float32),
        pltpu.VMEM((NCH, CH), jnp.float32),
    ],
)
def _gat_att(src_hbm, dst_hbm, e_hbm, s_hbm, d_hbm,
             exh_hbm, denp_hbm,
             s_loc, d_loc, denom_loc, src_loc, dst_loc, e_loc, ex_loc):
    cid = lax.axis_index("c")
    sid = lax.axis_index("s")
    wid = cid * 16 + sid
    z16 = jnp.zeros((16,), jnp.float32)

    def _zd(i, _):
        denom_loc[pl.ds(i * 16, 16)] = z16
        return 0
    lax.fori_loop(0, NP_ // 16, _zd, 0)

    pltpu.sync_copy(s_hbm, s_loc)
    pltpu.sync_copy(d_hbm, d_loc)
    pltpu.sync_copy(src_hbm.at[wid], src_loc)
    pltpu.sync_copy(dst_hbm.at[wid], dst_loc)
    pltpu.sync_copy(e_hbm.at[wid], e_loc)

    iota = lax.iota(jnp.int32, 16)
    ebase = wid * EPT

    def _att(t, _):
        c = t // (CH // 16)
        j = t % (CH // 16)
        si = src_loc[c, pl.ds(j * 16, 16)]
        di = dst_loc[c, pl.ds(j * 16, 16)]
        ev = e_loc[c, pl.ds(j * 16, 16)]
        sv = plsc.load_gather(s_loc, [si])
        dv = plsc.load_gather(d_loc, [di])
        a = sv + dv + ev
        a = jnp.where(a >= 0.0, a, 0.2 * a)
        ex = jnp.exp(a)
        gid = ebase + t * 16 + iota
        ex = jnp.where(gid < E, ex, 0.0)
        ex_loc[c, pl.ds(j * 16, 16)] = ex
        plsc.addupdate_scatter(denom_loc, [di], ex)
        return 0
    lax.fori_loop(0, EPT // 16, _att, 0)

    pltpu.sync_copy(ex_loc, exh_hbm.at[wid])
    pltpu.sync_copy(denom_loc, denp_hbm.at[wid])


GC = 8
TOTCH = EP // CH
C0CH = 40
C1CH = (TOTCH - 16 * C0CH) // 16


def _make_gat_rows(D):
    nv = D // 16

    @functools.partial(
        pl.kernel,
        out_type=jax.ShapeDtypeStruct((2, NP_, D), jnp.float32),
        mesh=_sc_mesh,
        compiler_params=pltpu.CompilerParams(needs_layout_passes=False),
        scratch_types=[
            pltpu.VMEM((GC, CH), jnp.int32),
            pltpu.VMEM((GC, CH), jnp.int32),
            pltpu.VMEM((GC, CH), jnp.float32),
            pltpu.VMEM((CH, D), jnp.float32),
            pltpu.VMEM((CH, D), jnp.float32),
            pltpu.VMEM_SHARED((NP_, D), jnp.float32),
            pltpu.SemaphoreType.DMA,
            pltpu.SemaphoreType.DMA,
            pltpu.SemaphoreType.DMA,
            pltpu.SemaphoreType.DMA,
        ],
    )
    def rows_kernel(src_hbm, dst_hbm, exh_hbm, h_hbm,
                    outp_hbm,
                    src_g, dst_g, ex_g, rows0, rows1, out_sh,
                    sem0, sem1, sems0, sems1):
        cid = lax.axis_index("c")
        sid = lax.axis_index("s")
        cstart = jnp.where(cid == 0, sid * C0CH, 16 * C0CH + sid * C1CH)
        ngrp = jnp.where(cid == 0, C0CH // GC, C1CH // GC)
        z16 = jnp.zeros((16,), jnp.float32)

        def _zr(i, _):
            r = i // nv
            c = i % nv
            rows0[r, pl.ds(c * 16, 16)] = z16
            return 0
        lax.fori_loop(0, CH * nv, _zr, 0)

        def _zs(i, _):
            pltpu.sync_copy(rows0, out_sh.at[pl.ds(sid * RPT + i * CH, CH)])
            return 0
        lax.fori_loop(0, RPT // CH, _zs, 0)
        plsc.subcore_barrier()

        def _scale(buf, k):
            def _sblk(j, _):
                w16 = ex_g[k, pl.ds(j * 16, 16)]
                base = j * 16
                for r in range(16):
                    w = w16.at[jnp.full((16,), r, jnp.int32)].get(
                        mode="promise_in_bounds")
                    i = base + r
                    for v in range(nv):
                        buf[i, pl.ds(v * 16, 16)] = (
                            buf[i, pl.ds(v * 16, 16)] * w)
                return 0
            lax.fori_loop(0, CH // 16, _sblk, 0)

        def _group(g, _):
            pltpu.sync_copy(src_hbm.at[pl.ds(cstart + g * GC, GC)], src_g)
            pltpu.sync_copy(dst_hbm.at[pl.ds(cstart + g * GC, GC)], dst_g)
            pltpu.sync_copy(exh_hbm.at[pl.ds(cstart + g * GC, GC)], ex_g)

            def _pair(cc, _):
                a = 2 * cc
                b = 2 * cc + 1
                cpa = pltpu.async_copy(h_hbm.at[src_g.at[a]], rows0, sem0)
                cpb = pltpu.async_copy(h_hbm.at[src_g.at[b]], rows1, sem1)
                cpa.wait()
                _scale(rows0, a)
                spa = pltpu.async_copy(rows0, out_sh.at[dst_g.at[a]],
                                       sems0, add=True)
                cpb.wait()
                _scale(rows1, b)
                spb = pltpu.async_copy(rows1, out_sh.at[dst_g.at[b]],
                                       sems1, add=True)
                spa.wait()
                spb.wait()
                return 0
            lax.fori_loop(0, GC // 2, _pair, 0)
            return 0
        lax.fori_loop(0, ngrp, _group, 0)

        plsc.subcore_barrier()

        def _wb(i, _):
            pltpu.sync_copy(out_sh.at[pl.ds(sid * RPT + i * CH, CH)], rows0)
            pltpu.sync_copy(rows0, outp_hbm.at[cid, pl.ds(sid * RPT + i * CH, CH)])
            return 0
        lax.fori_loop(0, RPT // CH, _wb, 0)

    return rows_kernel


_rows128 = _make_gat_rows(DH)


def kernel(x, edge_index, batch, edge_attr,
           W1, a_src1, a_dst1, We1, a_e1, b1,
           W2, a_src2, a_dst2, We2, a_e2, b2,
           Wl, bl, Wd1, bd1, Wd2, bd2):
    srcp = jnp.pad(edge_index[0], (0, EP - E)).reshape(NTILES, NCH, CH)
    dstp = jnp.pad(edge_index[1], (0, EP - E)).reshape(NTILES, NCH, CH)
    eap = jnp.pad(edge_attr, ((0, EP - E), (0, 0)))
    xp = jnp.pad(x, ((0, NP_ - N), (0, 0)))
    batchp = jnp.pad(batch, (0, NP_ - N), constant_values=G)
    batchr = batchp.reshape(NP_ // BN, 1, BN)

    h1, ssd1 = _node1(xp, W1, a_src1.reshape(1, DH), a_dst1.reshape(1, DH))
    e12 = _edgefeat(eap, We1, a_e1.reshape(1, DH), We2, a_e2.reshape(1, DL))[0]
    e1p = e12[0].reshape(NTILES, NCH, CH)
    e2p = e12[1].reshape(NTILES, NCH, CH)

    srcf = srcp.reshape(TOTCH, CH)
    dstf = dstp.reshape(TOTCH, CH)
    ex1, den1p = _gat_att(srcp, dstp, e1p, ssd1[0], ssd1[1])
    out1p = _rows128(srcf, dstf, ex1.reshape(TOTCH, CH), h1)
    h2, ssd2 = _combine2(out1p, den1p, b1.reshape(1, DH), W2,
                         a_src2.reshape(1, DL), a_dst2.reshape(1, DL))
    ex2, den2p = _gat_att(srcp, dstp, e2p, ssd2[0], ssd2[1])
    out2p = _rows128(srcf, dstf, ex2.reshape(TOTCH, CH), h2)
    z, xh, ge = _final(out2p, den2p, b2.reshape(1, DL), Wl, bl.reshape(1, DL),
                       batchr, Wd1, bd1.reshape(1, DH), Wd2, bd2.reshape(1, DOUT))
    return xh[:N], z[:N], ge


# speedup vs baseline: 1.2803x; 1.2803x over previous
import functools

import jax
import jax.numpy as jnp
from jax import lax
from jax.experimental import pallas as pl
from jax.experimental.pallas import tpu as pltpu
from jax.experimental.pallas import tpu_sc as plsc

N = 10000
E = 320000
DIN = 128
DH = 128
DL = 64
DOUT = 128
DE = 16
G = 64

NP_ = 10240
EP = 327680
NTILES = 32
EPT = EP // NTILES
CH = 128
NCH = EPT // CH
BN = 1024
RPT = NP_ // 16


def _node1_body(x_ref, w_ref, asrc_ref, adst_ref, h_ref, ssd_ref):
    xw = jnp.dot(x_ref[...], w_ref[...], preferred_element_type=jnp.float32)
    h_ref[...] = xw
    s = jnp.sum(xw * asrc_ref[...], axis=1).reshape(1, BN)
    d = jnp.sum(xw * adst_ref[...], axis=1).reshape(1, BN)
    ssd_ref[...] = jnp.concatenate(
        [s, d, jnp.zeros((6, BN), jnp.float32)], axis=0)


_node1 = pl.pallas_call(
    _node1_body,
    grid=(NP_ // BN,),
    in_specs=[
        pl.BlockSpec((BN, DIN), lambda j: (j, 0)),
        pl.BlockSpec((DIN, DH), lambda j: (0, 0)),
        pl.BlockSpec((1, DH), lambda j: (0, 0)),
        pl.BlockSpec((1, DH), lambda j: (0, 0)),
    ],
    out_specs=[
        pl.BlockSpec((BN, DH), lambda j: (j, 0)),
        pl.BlockSpec((8, BN), lambda j: (0, j)),
    ],
    out_shape=[
        jax.ShapeDtypeStruct((NP_, DH), jnp.float32),
        jax.ShapeDtypeStruct((8, NP_), jnp.float32),
    ],
)

BE = 4096


def _edgefeat_body(ea_ref, we1_ref, ae1_ref, we2_ref, ae2_ref, e12_ref):
    ea = ea_ref[...]
    ve1 = jnp.sum(we1_ref[...] * ae1_ref[...], axis=1)
    ve2 = jnp.sum(we2_ref[...] * ae2_ref[...], axis=1)
    e1 = jnp.sum(ea * ve1[None, :], axis=1).reshape(1, BE)
    e2 = jnp.sum(ea * ve2[None, :], axis=1).reshape(1, BE)
    e12_ref[...] = jnp.concatenate(
        [e1, e2, jnp.zeros((6, BE), jnp.float32)], axis=0)


_edgefeat = pl.pallas_call(
    _edgefeat_body,
    grid=(EP // BE,),
    in_specs=[
        pl.BlockSpec((BE, DE), lambda j: (j, 0)),
        pl.BlockSpec((DE, DH), lambda j: (0, 0)),
        pl.BlockSpec((1, DH), lambda j: (0, 0)),
        pl.BlockSpec((DE, DL), lambda j: (0, 0)),
        pl.BlockSpec((1, DL), lambda j: (0, 0)),
    ],
    out_specs=[pl.BlockSpec((8, BE), lambda j: (0, j))],
    out_shape=[jax.ShapeDtypeStruct((8, EP), jnp.float32)],
)


def _combine2_body(op_ref, dp_ref, b1_ref, w2_ref, as2_ref, ad2_ref,
                   h2_ref, ssd_ref):
    agg = op_ref[0] + op_ref[1]
    den = jnp.sum(dp_ref[...], axis=0)
    hr = jnp.maximum(agg / (den[:, None] + 1e-16) + b1_ref[...], 0.0)
    h2 = jnp.dot(hr, w2_ref[...], preferred_element_type=jnp.float32)
    h2_ref[...] = jnp.concatenate(
        [h2, jnp.zeros((BN, DH - DL), jnp.float32)], axis=1)
    s = jnp.sum(h2 * as2_ref[...], axis=1).reshape(1, BN)
    d = jnp.sum(h2 * ad2_ref[...], axis=1).reshape(1, BN)
    ssd_ref[...] = jnp.concatenate(
        [s, d, jnp.zeros((6, BN), jnp.float32)], axis=0)


_combine2 = pl.pallas_call(
    _combine2_body,
    grid=(NP_ // BN,),
    in_specs=[
        pl.BlockSpec((2, BN, DH), lambda j: (0, j, 0)),
        pl.BlockSpec((NTILES, BN), lambda j: (0, j)),
        pl.BlockSpec((1, DH), lambda j: (0, 0)),
        pl.BlockSpec((DH, DL), lambda j: (0, 0)),
        pl.BlockSpec((1, DL), lambda j: (0, 0)),
        pl.BlockSpec((1, DL), lambda j: (0, 0)),
    ],
    out_specs=[
        pl.BlockSpec((BN, DH), lambda j: (j, 0)),
        pl.BlockSpec((8, BN), lambda j: (0, j)),
    ],
    out_shape=[
        jax.ShapeDtypeStruct((NP_, DH), jnp.float32),
        jax.ShapeDtypeStruct((8, NP_), jnp.float32),
    ],
)


def _final_body(op_ref, dp_ref, b2_ref, wl_ref, bl_ref, batch_ref,
                wd1_ref, bd1_ref, wd2_ref, bd2_ref,
                z_ref, xh_ref, ge_ref, pool_ref):
    j = pl.program_id(0)
    agg = (op_ref[0] + op_ref[1])[:, 0:DL]
    den = jnp.sum(dp_ref[...], axis=0)
    zin = agg / (den[:, None] + 1e-16) + b2_ref[...]
    z = jnp.dot(zin, wl_ref[...], preferred_element_type=jnp.float32) + bl_ref[...]
    z_ref[...] = z
    b = batch_ref[0, 0, :]
    onehot = (b[:, None] == lax.broadcasted_iota(jnp.int32, (1, G), 1)
              ).astype(jnp.float32)
    gs = lax.dot_general(onehot, z, (((0,), (0,)), ((), ())),
                         preferred_element_type=jnp.float32)
    cnt = jnp.sum(onehot, axis=0).reshape(G, 1)

    @pl.when(j == 0)
    def _():
        pool_ref[...] = jnp.zeros((G, 128), jnp.float32)

    pool_ref[:, 0:DL] = pool_ref[:, 0:DL] + gs
    pool_ref[:, DL:DL + 1] = pool_ref[:, DL:DL + 1] + cnt

    hd = jnp.maximum(
        jnp.dot(z, wd1_ref[...], preferred_element_type=jnp.float32)
        + bd1_ref[...], 0.0)
    xh_ref[...] = (jnp.dot(hd, wd2_ref[...], preferred_element_type=jnp.float32)
                   + bd2_ref[...])

    @pl.when(j == NP_ // BN - 1)
    def _():
        ge_ref[...] = pool_ref[:, 0:DL] / jnp.maximum(
            pool_ref[:, DL:DL + 1], 1.0)


_final = pl.pallas_call(
    _final_body,
    grid=(NP_ // BN,),
    in_specs=[
        pl.BlockSpec((2, BN, DH), lambda j: (0, j, 0)),
        pl.BlockSpec((NTILES, BN), lambda j: (0, j)),
        pl.BlockSpec((1, DL), lambda j: (0, 0)),
        pl.BlockSpec((DL, DL), lambda j: (0, 0)),
        pl.BlockSpec((1, DL), lambda j: (0, 0)),
        pl.BlockSpec((1, 1, BN), lambda j: (j, 0, 0)),
        pl.BlockSpec((DL, DH), lambda j: (0, 0)),
        pl.BlockSpec((1, DH), lambda j: (0, 0)),
        pl.BlockSpec((DH, DOUT), lambda j: (0, 0)),
        pl.BlockSpec((1, DOUT), lambda j: (0, 0)),
    ],
    out_specs=[
        pl.BlockSpec((BN, DL), lambda j: (j, 0)),
        pl.BlockSpec((BN, DOUT), lambda j: (j, 0)),
        pl.BlockSpec((G, DL), lambda j: (0, 0)),
    ],
    out_shape=[
        jax.ShapeDtypeStruct((NP_, DL), jnp.float32),
        jax.ShapeDtypeStruct((NP_, DOUT), jnp.float32),
        jax.ShapeDtypeStruct((G, DL), jnp.float32),
    ],
    scratch_shapes=[pltpu.VMEM((G, 128), jnp.float32)],
)


_sc_mesh = plsc.VectorSubcoreMesh(core_axis_name="c", subcore_axis_name="s")


@functools.partial(
    pl.kernel,
    out_type=[
        jax.ShapeDtypeStruct((NTILES, NCH, CH), jnp.float32),
        jax.ShapeDtypeStruct((NTILES, NP_), jnp.float32),
    ],
    mesh=_sc_mesh,
    compiler_params=pltpu.CompilerParams(needs_layout_passes=False),
    scratch_types=[
        pltpu.VMEM((NP_,), jnp.float32),
        pltpu.VMEM((NP_,), jnp.float32),
        pltpu.VMEM((NP_,), jnp.float32),
        pltpu.VMEM((NCH, CH), jnp.int32),
        pltpu.VMEM((NCH, CH), jnp.int32),
        pltpu.VMEM((NCH, CH), jnp.float32),
        pltpu.VMEM((NCH, CH), jnp.float32),
    ],
)
def _gat_att(src_hbm, dst_hbm, e_hbm, s_hbm, d_hbm,
             exh_hbm, denp_hbm,
             s_loc, d_loc, denom_loc, src_loc, dst_loc, e_loc, ex_loc):
    cid = lax.axis_index("c")
    sid = lax.axis_index("s")
    wid = cid * 16 + sid
    z16 = jnp.zeros((16,), jnp.float32)

    def _zd(i, _):
        denom_loc[pl.ds(i * 16, 16)] = z16
        return 0
    lax.fori_loop(0, NP_ // 16, _zd, 0)

    pltpu.sync_copy(s_hbm, s_loc)
    pltpu.sync_copy(d_hbm, d_loc)
    pltpu.sync_copy(src_hbm.at[wid], src_loc)
    pltpu.sync_copy(dst_hbm.at[wid], dst_loc)
    pltpu.sync_copy(e_hbm.at[wid], e_loc)

    iota = lax.iota(jnp.int32, 16)
    ebase = wid * EPT

    def _att(t, _):
        c = t // (CH // 16)
        j = t % (CH // 16)
        si = src_loc[c, pl.ds(j * 16, 16)]
        di = dst_loc[c, pl.ds(j * 16, 16)]
        ev = e_loc[c, pl.ds(j * 16, 16)]
        sv = plsc.load_gather(s_loc, [si])
        dv = plsc.load_gather(d_loc, [di])
        a = sv + dv + ev
        a = jnp.where(a >= 0.0, a, 0.2 * a)
        ex = jnp.exp(a)
        gid = ebase + t * 16 + iota
        ex = jnp.where(gid < E, ex, 0.0)
        ex_loc[c, pl.ds(j * 16, 16)] = ex
        plsc.addupdate_scatter(denom_loc, [di], ex)
        return 0
    lax.fori_loop(0, EPT // 16, _att, 0)

    pltpu.sync_copy(ex_loc, exh_hbm.at[wid])
    pltpu.sync_copy(denom_loc, denp_hbm.at[wid])


GC = 8
TOTCH = EP // CH
C0CH = 120
C1CH = (TOTCH - 16 * C0CH) // 16


def _make_gat_rows(D):
    nv = D // 16

    @functools.partial(
        pl.kernel,
        out_type=jax.ShapeDtypeStruct((2, NP_, D), jnp.float32),
        mesh=_sc_mesh,
        compiler_params=pltpu.CompilerParams(needs_layout_passes=False),
        scratch_types=[
            pltpu.VMEM((GC, CH), jnp.int32),
            pltpu.VMEM((GC, CH), jnp.int32),
            pltpu.VMEM((GC, CH), jnp.float32),
            pltpu.VMEM((CH, D), jnp.float32),
            pltpu.VMEM((CH, D), jnp.float32),
            pltpu.VMEM_SHARED((NP_, D), jnp.float32),
            pltpu.SemaphoreType.DMA,
            pltpu.SemaphoreType.DMA,
            pltpu.SemaphoreType.DMA,
            pltpu.SemaphoreType.DMA,
        ],
    )
    def rows_kernel(src_hbm, dst_hbm, exh_hbm, h_hbm,
                    outp_hbm,
                    src_g, dst_g, ex_g, rows0, rows1, out_sh,
                    sem0, sem1, sems0, sems1):
        cid = lax.axis_index("c")
        sid = lax.axis_index("s")
        cstart = jnp.where(cid == 0, sid * C0CH, 16 * C0CH + sid * C1CH)
        ngrp = jnp.where(cid == 0, C0CH // GC, C1CH // GC)
        z16 = jnp.zeros((16,), jnp.float32)

        def _zr(i, _):
            r = i // nv
            c = i % nv
            rows0[r, pl.ds(c * 16, 16)] = z16
            return 0
        lax.fori_loop(0, CH * nv, _zr, 0)

        def _zs(i, _):
            pltpu.sync_copy(rows0, out_sh.at[pl.ds(sid * RPT + i * CH, CH)])
            return 0
        lax.fori_loop(0, RPT // CH, _zs, 0)
        plsc.subcore_barrier()

        def _scale(buf, k):
            def _sblk(j, _):
                w16 = ex_g[k, pl.ds(j * 16, 16)]
                base = j * 16
                for r in range(16):
                    w = w16.at[jnp.full((16,), r, jnp.int32)].get(
                        mode="promise_in_bounds")
                    i = base + r
                    for v in range(nv):
                        buf[i, pl.ds(v * 16, 16)] = (
                            buf[i, pl.ds(v * 16, 16)] * w)
                return 0
            lax.fori_loop(0, CH // 16, _sblk, 0)

        def _group(g, _):
            pltpu.sync_copy(src_hbm.at[pl.ds(cstart + g * GC, GC)], src_g)
            pltpu.sync_copy(dst_hbm.at[pl.ds(cstart + g * GC, GC)], dst_g)
            pltpu.sync_copy(exh_hbm.at[pl.ds(cstart + g * GC, GC)], ex_g)

            def _pair(cc, _):
                a = 2 * cc
                b = 2 * cc + 1
                cpa = pltpu.async_copy(h_hbm.at[src_g.at[a]], rows0, sem0)
                cpb = pltpu.async_copy(h_hbm.at[src_g.at[b]], rows1, sem1)
                cpa.wait()
                _scale(rows0, a)
                spa = pltpu.async_copy(rows0, out_sh.at[dst_g.at[a]],
                                       sems0, add=True)
                cpb.wait()
                _scale(rows1, b)
                spb = pltpu.async_copy(rows1, out_sh.at[dst_g.at[b]],
                                       sems1, add=True)
                spa.wait()
                spb.wait()
                return 0
            lax.fori_loop(0, GC // 2, _pair, 0)
            return 0
        lax.fori_loop(0, ngrp, _group, 0)

        plsc.subcore_barrier()

        def _wb(i, _):
            pltpu.sync_copy(out_sh.at[pl.ds(sid * RPT + i * CH, CH)], rows0)
            pltpu.sync_copy(rows0, outp_hbm.at[cid, pl.ds(sid * RPT + i * CH, CH)])
            return 0
        lax.fori_loop(0, RPT // CH, _wb, 0)

    return rows_kernel


_rows128 = _make_gat_rows(DH)


def kernel(x, edge_index, batch, edge_attr,
           W1, a_src1, a_dst1, We1, a_e1, b1,
           W2, a_src2, a_dst2, We2, a_e2, b2,
           Wl, bl, Wd1, bd1, Wd2, bd2):
    srcp = jnp.pad(edge_index[0], (0, EP - E)).reshape(NTILES, NCH, CH)
    dstp = jnp.pad(edge_index[1], (0, EP - E)).reshape(NTILES, NCH, CH)
    eap = jnp.pad(edge_attr, ((0, EP - E), (0, 0)))
    xp = jnp.pad(x, ((0, NP_ - N), (0, 0)))
    batchp = jnp.pad(batch, (0, NP_ - N), constant_values=G)
    batchr = batchp.reshape(NP_ // BN, 1, BN)

    h1, ssd1 = _node1(xp, W1, a_src1.reshape(1, DH), a_dst1.reshape(1, DH))
    e12 = _edgefeat(eap, We1, a_e1.reshape(1, DH), We2, a_e2.reshape(1, DL))[0]
    e1p = e12[0].reshape(NTILES, NCH, CH)
    e2p = e12[1].reshape(NTILES, NCH, CH)

    srcf = srcp.reshape(TOTCH, CH)
    dstf = dstp.reshape(TOTCH, CH)
    ex1, den1p = _gat_att(srcp, dstp, e1p, ssd1[0], ssd1[1])
    out1p = _rows128(srcf, dstf, ex1.reshape(TOTCH, CH), h1)
    h2, ssd2 = _combine2(out1p, den1p, b1.reshape(1, DH), W2,
                         a_src2.reshape(1, DL), a_dst2.reshape(1, DL))
    ex2, den2p = _gat_att(srcp, dstp, e2p, ssd2[0], ssd2[1])
    out2p = _rows128(srcf, dstf, ex2.reshape(TOTCH, CH), h2)
    z, xh, ge = _final(out2p, den2p, b2.reshape(1, DL), Wl, bl.reshape(1, DL),
                       batchr, Wd1, bd1.reshape(1, DH), Wd2, bd2.reshape(1, DOUT))
    return xh[:N], z[:N], ge
